# speedup vs baseline: 4.8689x; 4.8689x over previous
import functools

import jax
import jax.numpy as jnp
from jax import lax
from jax.experimental import pallas as pl
from jax.experimental.pallas import tpu as pltpu
from jax.experimental.pallas import tpu_sc as plsc

B, T, C = 16, 4096, 256
G, V = 2, 512
DV = C // G
NTOK = B * T
TB = 1024
GRID = NTOK // TB

NC, NS, L = 2, 16, 16
NW = NC * NS
TPW = NTOK // NW
CB = 128
NCH = TPW // CB


def _tables_body(cb_ref, wq_ref, bq_ref, m0_ref, m1_ref):
    wq = wq_ref[...]
    m0_ref[...] = (
        jnp.dot(cb_ref[0], wq[:DV, :], preferred_element_type=jnp.float32)
        + bq_ref[...]
    )
    m1_ref[...] = jnp.dot(cb_ref[1], wq[DV:, :], preferred_element_type=jnp.float32)


def _build_tables(codebook, Wq, bq):
    return pl.pallas_call(
        _tables_body,
        out_shape=(
            jax.ShapeDtypeStruct((V, C), jnp.float32),
            jax.ShapeDtypeStruct((V, C), jnp.float32),
        ),
    )(codebook, Wq, bq.reshape(1, C))


def _row_argmax(l):
    m = jnp.max(l, axis=1, keepdims=True)
    ii = lax.broadcasted_iota(jnp.int32, l.shape, 1)
    return jnp.min(jnp.where(l == m, ii, V), axis=1)


def _stage1_body(z_ref, wp_ref, bp_ref, i0_ref, i1_ref):
    logits = (
        jnp.dot(z_ref[...], wp_ref[...], preferred_element_type=jnp.float32)
        + bp_ref[...]
    )
    i0_ref[...] = _row_argmax(logits[:, :V]).reshape(1, 1, TB)
    i1_ref[...] = _row_argmax(logits[:, V:]).reshape(1, 1, TB)


def _compute_indices(flat_z, W_proj, b_proj):
    idx_shape = jax.ShapeDtypeStruct((GRID, 1, TB), jnp.int32)
    i0, i1 = pl.pallas_call(
        _stage1_body,
        grid=(GRID,),
        in_specs=[
            pl.BlockSpec((TB, C), lambda i: (i, 0)),
            pl.BlockSpec((C, G * V), lambda i: (0, 0)),
            pl.BlockSpec((1, G * V), lambda i: (0, 0)),
        ],
        out_specs=(
            pl.BlockSpec((1, 1, TB), lambda i: (i, 0, 0)),
            pl.BlockSpec((1, 1, TB), lambda i: (i, 0, 0)),
        ),
        out_shape=(idx_shape, idx_shape),
    )(flat_z, W_proj, b_proj.reshape(1, G * V))
    return i0.reshape(NTOK), i1.reshape(NTOK)


def _sc_gather_body(m0_hbm, m1_hbm, idx0_hbm, idx1_hbm, out_hbm,
                    i0v, i1v, r0v, r1v, sem0, sem1):
    wid = lax.axis_index("s") * NC + lax.axis_index("c")
    base = wid * TPW

    def chunk(ci, carry):
        off = base + ci * CB
        pltpu.sync_copy(idx0_hbm.at[pl.ds(off, CB)], i0v)
        pltpu.sync_copy(idx1_hbm.at[pl.ds(off, CB)], i1v)
        c0 = pltpu.async_copy(m0_hbm.at[i0v], r0v, sem0)
        c1 = pltpu.async_copy(m1_hbm.at[i1v], r1v, sem1)
        c0.wait()
        c1.wait()

        def tok(t, c2):
            for j in range(C // L):
                sl = pl.ds(j * L, L)
                r0v[t, sl] = r0v[t, sl] + r1v[t, sl]
            return c2

        lax.fori_loop(0, CB, tok, 0)
        pltpu.sync_copy(r0v, out_hbm.at[pl.ds(off, CB)])
        return carry

    lax.fori_loop(0, NCH, chunk, 0)


def _sc_gather(m0, m1, idx0, idx1):
    mesh = plsc.VectorSubcoreMesh(core_axis_name="c", subcore_axis_name="s")
    fn = functools.partial(
        pl.kernel,
        mesh=mesh,
        out_type=jax.ShapeDtypeStruct((NTOK, C), jnp.float32),
        scratch_types=[
            pltpu.VMEM((CB,), jnp.int32),
            pltpu.VMEM((CB,), jnp.int32),
            pltpu.VMEM((CB, C), jnp.float32),
            pltpu.VMEM((CB, C), jnp.float32),
            pltpu.SemaphoreType.DMA,
            pltpu.SemaphoreType.DMA,
        ],
    )(_sc_gather_body)
    return fn(m0, m1, idx0, idx1)


def kernel(z, W_proj, b_proj, codebook, Wq, bq):
    flat_z = z.reshape(NTOK, C)
    m0, m1 = _build_tables(codebook, Wq, bq)
    idx0, idx1 = _compute_indices(flat_z, W_proj, b_proj)
    q = _sc_gather(m0, m1, idx0, idx1)
    return q.reshape(B, T, C)


# speedup vs baseline: 6.5143x; 1.3379x over previous
import functools

import jax
import jax.numpy as jnp
from jax import lax
from jax.experimental import pallas as pl
from jax.experimental.pallas import tpu as pltpu
from jax.experimental.pallas import tpu_sc as plsc

B, T, C = 16, 4096, 256
G, V = 2, 512
DV = C // G
NTOK = B * T
TB = 1024
GRID = NTOK // TB

NC, NS, L = 2, 16, 16
NW = NC * NS
TPW = NTOK // NW
CB = 64
NCH = TPW // CB


def _tables_body(cb_ref, wq_ref, bq_ref, m0_ref, m1_ref):
    wq = wq_ref[...]
    m0_ref[...] = (
        jnp.dot(cb_ref[0], wq[:DV, :], preferred_element_type=jnp.float32)
        + bq_ref[...]
    )
    m1_ref[...] = jnp.dot(cb_ref[1], wq[DV:, :], preferred_element_type=jnp.float32)


def _build_tables(codebook, Wq, bq):
    return pl.pallas_call(
        _tables_body,
        out_shape=(
            jax.ShapeDtypeStruct((V, C), jnp.float32),
            jax.ShapeDtypeStruct((V, C), jnp.float32),
        ),
    )(codebook, Wq, bq.reshape(1, C))


def _col_argmax(lt):
    m = jnp.max(lt, axis=0, keepdims=True)
    ii = lax.broadcasted_iota(jnp.int32, lt.shape, 0)
    return jnp.min(jnp.where(lt == m, ii, V), axis=0)


def _stage1_body(z_ref, wp_ref, bp_ref, i0_ref, i1_ref):
    logits_t = lax.dot_general(
        wp_ref[...], z_ref[...],
        dimension_numbers=(((0,), (1,)), ((), ())),
        preferred_element_type=jnp.float32,
    ) + bp_ref[...]
    i0_ref[...] = _col_argmax(logits_t[:V, :]).reshape(1, 1, TB)
    i1_ref[...] = _col_argmax(logits_t[V:, :]).reshape(1, 1, TB)


def _compute_indices(flat_z, W_proj, b_proj):
    idx_shape = jax.ShapeDtypeStruct((GRID, 1, TB), jnp.int32)
    i0, i1 = pl.pallas_call(
        _stage1_body,
        grid=(GRID,),
        in_specs=[
            pl.BlockSpec((TB, C), lambda i: (i, 0)),
            pl.BlockSpec((C, G * V), lambda i: (0, 0)),
            pl.BlockSpec((G * V, 1), lambda i: (0, 0)),
        ],
        out_specs=(
            pl.BlockSpec((1, 1, TB), lambda i: (i, 0, 0)),
            pl.BlockSpec((1, 1, TB), lambda i: (i, 0, 0)),
        ),
        out_shape=(idx_shape, idx_shape),
    )(flat_z, W_proj, b_proj.reshape(G * V, 1))
    return i0.reshape(NTOK), i1.reshape(NTOK)


def _sc_gather_body(m0_hbm, m1_hbm, idx0_hbm, idx1_hbm, out_hbm,
                    i0v, i1v,
                    r0a, r1a, r0b, r1b, r0c, r1c,
                    isem, gsa, gsb, gsc, wsa, wsb, wsc):
    wid = lax.axis_index("s") * NC + lax.axis_index("c")
    base = wid * TPW

    ic0 = pltpu.async_copy(idx0_hbm.at[pl.ds(base, TPW)], i0v, isem)
    ic1 = pltpu.async_copy(idx1_hbm.at[pl.ds(base, TPW)], i1v, isem)
    ic0.wait()
    ic1.wait()

    bufs = ((r0a, r1a, gsa, wsa), (r0b, r1b, gsb, wsb), (r0c, r1c, gsc, wsc))

    def fire(ci):
        r0, r1, gs, _ = bufs[ci % 3]
        s = ci * CB
        h0 = pltpu.async_copy(m0_hbm.at[i0v.at[pl.ds(s, CB)]], r0, gs)
        h1 = pltpu.async_copy(m1_hbm.at[i1v.at[pl.ds(s, CB)]], r1, gs)
        return (h0, h1)

    wb = [None] * NCH
    gh = [None] * NCH
    gh[0] = fire(0)
    for ci in range(NCH):
        r0, r1, gs, ws = bufs[ci % 3]
        if ci + 1 < NCH:
            if ci + 1 >= 3:
                wb[ci - 2].wait()
            gh[ci + 1] = fire(ci + 1)
        gh[ci][0].wait()
        gh[ci][1].wait()

        def tok(t, c2):
            for j in range(C // L):
                sl = pl.ds(j * L, L)
                plsc.addupdate(r0.at[t, sl], r1[t, sl])
            return c2

        lax.fori_loop(0, CB, tok, 0)
        wb[ci] = pltpu.async_copy(r0, out_hbm.at[pl.ds(base + ci * CB, CB)], ws)
    for ci in range(max(0, NCH - 3), NCH):
        wb[ci].wait()


def _sc_gather(m0, m1, idx0, idx1):
    mesh = plsc.VectorSubcoreMesh(core_axis_name="c", subcore_axis_name="s")
    fn = functools.partial(
        pl.kernel,
        mesh=mesh,
        out_type=jax.ShapeDtypeStruct((NTOK, C), jnp.float32),
        scratch_types=[
            pltpu.VMEM((TPW,), jnp.int32),
            pltpu.VMEM((TPW,), jnp.int32),
            pltpu.VMEM((CB, C), jnp.float32),
            pltpu.VMEM((CB, C), jnp.float32),
            pltpu.VMEM((CB, C), jnp.float32),
            pltpu.VMEM((CB, C), jnp.float32),
            pltpu.VMEM((CB, C), jnp.float32),
            pltpu.VMEM((CB, C), jnp.float32),
            pltpu.SemaphoreType.DMA,
            pltpu.SemaphoreType.DMA,
            pltpu.SemaphoreType.DMA,
            pltpu.SemaphoreType.DMA,
            pltpu.SemaphoreType.DMA,
            pltpu.SemaphoreType.DMA,
            pltpu.SemaphoreType.DMA,
        ],
    )(_sc_gather_body)
    return fn(m0, m1, idx0, idx1)


def kernel(z, W_proj, b_proj, codebook, Wq, bq):
    flat_z = z.reshape(NTOK, C)
    m0, m1 = _build_tables(codebook, Wq, bq)
    idx0, idx1 = _compute_indices(flat_z, W_proj, b_proj)
    q = _sc_gather(m0, m1, idx0, idx1)
    return q.reshape(B, T, C)
